# baseline (speedup 1.0000x reference)
import functools

import jax
import jax.numpy as jnp
from jax import lax
from jax.experimental import pallas as pl
from jax.experimental.pallas import tpu as pltpu
from jax.experimental.pallas import tpu_sc as plsc

_N_CELLS = 1024
_N_GENES_OI = 4096
_N_IN = 16
_N_OUT = 16
_D = _N_IN * _N_OUT


def _make_sc_gather(n_rows, d, num_workers, rows_per_worker, num_cores):
    def body(table_hbm, idx_hbm, out_hbm, idx_v, rows_v, sem):
        wid = lax.axis_index("s") * num_cores + lax.axis_index("c")
        base = wid * rows_per_worker
        pltpu.sync_copy(idx_hbm.at[pl.ds(base, rows_per_worker)], idx_v)
        pltpu.async_copy(table_hbm.at[idx_v], rows_v, sem).wait()
        pltpu.sync_copy(rows_v, out_hbm.at[pl.ds(base, rows_per_worker)])

    return pl.kernel(
        body,
        out_type=jax.ShapeDtypeStruct((n_rows, d), jnp.float32),
        mesh=plsc.VectorSubcoreMesh(core_axis_name="c", subcore_axis_name="s"),
        scratch_types=[
            pltpu.VMEM((rows_per_worker,), jnp.int32),
            pltpu.VMEM((rows_per_worker, d), jnp.float32),
            pltpu.SemaphoreType.DMA,
        ],
    )


def _matmul_body(x_ref, w_ref, b_ref, o_ref):
    k = pl.program_id(0)
    xb = x_ref[...].astype(jnp.bfloat16)
    wb = w_ref[...].astype(jnp.bfloat16)
    acc = lax.dot_general(
        xb, wb, (((1,), (0,)), ((), ())), preferred_element_type=jnp.float32
    )

    @pl.when(k == 0)
    def _():
        o_ref[...] = acc + b_ref[...]

    @pl.when(k > 0)
    def _():
        o_ref[...] += acc


def kernel(cellgene_embedding, genes_oi, weight1, bias1):
    n_cells, n_genes_oi, n_in = cellgene_embedding.shape
    n_out = weight1.shape[2]
    d = n_in * n_out

    info = plsc.get_sparse_core_info()
    num_workers = info.num_cores * info.num_subcores
    rows_per_worker = n_genes_oi // num_workers

    table2d = weight1.reshape(weight1.shape[0], d)
    gather = _make_sc_gather(n_genes_oi, d, num_workers, rows_per_worker,
                             info.num_cores)
    w_rows = gather(table2d, genes_oi.astype(jnp.int32))

    big_k = n_genes_oi * n_in
    x2 = cellgene_embedding.reshape(n_cells, big_k)
    w2 = w_rows.reshape(big_k, n_out)
    bias2 = bias1.reshape(1, n_out)

    bk = 4096
    grid = (big_k // bk,)
    out = pl.pallas_call(
        _matmul_body,
        grid=grid,
        in_specs=[
            pl.BlockSpec((n_cells, bk), lambda k: (0, k)),
            pl.BlockSpec((bk, n_out), lambda k: (k, 0)),
            pl.BlockSpec((1, n_out), lambda k: (0, 0)),
        ],
        out_specs=pl.BlockSpec((n_cells, n_out), lambda k: (0, 0)),
        out_shape=jax.ShapeDtypeStruct((n_cells, n_out), jnp.float32),
        compiler_params=pltpu.CompilerParams(
            dimension_semantics=("arbitrary",),
        ),
    )(x2, w2, bias2)
    return out
